# baseline (speedup 1.0000x reference)
import functools

import jax
import jax.numpy as jnp
from jax import lax
from jax.experimental import pallas as pl
from jax.experimental.pallas import tpu as pltpu
from jax.experimental.pallas import tpu_sc as plsc

N = 10000
F = 128
C = 16
S2 = 2 * C
NC = 2
NS = 16
NW = NC * NS
CH = 128
NPAD = 10240
RPW = NPAD // NS
BR = 2000
SK = (48, 111)


NBUF = 3


def _spmm_body(src_hbm, dst_hbm, wgt_hbm, h_hbm, out_hbm,
               src0, dst0, wgt0, msg0,
               src1, dst1, wgt1, msg1,
               src2, dst2, wgt2, msg2,
               zbuf_v, acc_sh,
               isem0, isem1, isem2, gsem0, gsem1, gsem2,
               ssem0, ssem1, ssem2):
    cid = lax.axis_index("c")
    sid = lax.axis_index("s")
    base = jnp.where(cid == 0, sid * SK[0], NS * SK[0] + sid * SK[1])
    ngr = jnp.where(cid == 0, SK[0] // NBUF, SK[1] // NBUF)

    srcs = (src0, src1, src2)
    dsts = (dst0, dst1, dst2)
    wgts = (wgt0, wgt1, wgt2)
    msgs = (msg0, msg1, msg2)
    isems = (isem0, isem1, isem2)
    gsems = (gsem0, gsem1, gsem2)
    ssems = (ssem0, ssem1, ssem2)

    zeros16 = jnp.zeros((16,), jnp.float32)

    def zrow(i, carry):
        zbuf_v[i, pl.ds(0, 16)] = zeros16
        zbuf_v[i, pl.ds(16, 16)] = zeros16
        return carry

    lax.fori_loop(0, RPW, zrow, 0, unroll=8)
    pltpu.sync_copy(zbuf_v, acc_sh.at[pl.ds(sid * RPW, RPW)])
    plsc.subcore_barrier()

    def fire_idx(j, b):
        pltpu.async_copy(src_hbm.at[base + j], srcs[b], isems[b])
        pltpu.async_copy(dst_hbm.at[base + j], dsts[b], isems[b])
        pltpu.async_copy(wgt_hbm.at[base + j], wgts[b], isems[b])

    def wait_idx(j, b):
        pltpu.make_async_copy(src_hbm.at[base + j], srcs[b], isems[b]).wait()
        pltpu.make_async_copy(dst_hbm.at[base + j], dsts[b], isems[b]).wait()
        pltpu.make_async_copy(wgt_hbm.at[base + j], wgts[b], isems[b]).wait()

    def wait_scatter(b):
        pltpu.make_async_copy(msgs[b], acc_sh.at[dsts[b]], ssems[b]).wait()

    def scale(b):
        wv_ = wgts[b]
        mv = msgs[b]

        def scale16(k, c2):
            wvv = wv_[pl.ds(k * 16, 16)]
            base = k * 16
            for l in range(16):
                w = wvv[l]
                mv[base + l, pl.ds(0, 16)] = mv[base + l, pl.ds(0, 16)] * w
                mv[base + l, pl.ds(16, 16)] = mv[base + l, pl.ds(16, 16)] * w
            return c2

        lax.fori_loop(0, CH // 16, scale16, 0)

    fire_idx(0, 0)
    fire_idx(1, 1)

    def group(g, carry):
        for b in range(NBUF):
            j = NBUF * g + b
            bp = (b + 2) % NBUF
            wait_idx(j, b)
            gg = pltpu.async_copy(h_hbm.at[srcs[b]], msgs[b], gsems[b])
            if b == 0:
                @pl.when(g > 0)
                def _():
                    wait_scatter(bp)
                fire_idx(j + 2, bp)
            else:
                wait_scatter(bp)

                @pl.when(g < ngr - 1)
                def _():
                    fire_idx(j + 2, bp)
            gg.wait()
            scale(b)
            pltpu.async_copy(msgs[b], acc_sh.at[dsts[b]], ssems[b], add=True)
        return carry

    lax.fori_loop(0, ngr, group, 0)
    wait_scatter(NBUF - 1)
    plsc.subcore_barrier()
    pltpu.sync_copy(acc_sh.at[pl.ds(sid * RPW, RPW)],
                    out_hbm.at[cid, pl.ds(sid * RPW, RPW)])


def _make_spmm():
    return pl.kernel(
        _spmm_body,
        out_type=jax.ShapeDtypeStruct((NC, NPAD, S2), jnp.float32),
        mesh=plsc.VectorSubcoreMesh(core_axis_name="c", subcore_axis_name="s"),
        scratch_types=(
            [pltpu.VMEM((CH,), jnp.int32),
             pltpu.VMEM((CH,), jnp.int32),
             pltpu.VMEM((CH,), jnp.float32),
             pltpu.VMEM((CH, S2), jnp.float32)] * NBUF
            + [pltpu.VMEM((RPW, S2), jnp.float32),
               pltpu.VMEM_SHARED((NPAD, S2), jnp.float32)]
            + [pltpu.SemaphoreType.DMA] * (3 * NBUF)
        ),
        compiler_params=pltpu.CompilerParams(use_tc_tiling_on_sc=False),
    )


def _elu(v):
    return jnp.where(v > 0, v, jnp.exp(jnp.minimum(v, 0.0)) - 1.0)


def _proj_body(x_ref, wcat_ref, h_ref, skip_ref):
    hs = jnp.dot(x_ref[...], wcat_ref[...], preferred_element_type=jnp.float32)
    h_ref[...] = hs[:, :S2]
    skip_ref[...] = hs[:, S2:]


def _tc_proj(x, wcat):
    f = x.shape[1]
    return pl.pallas_call(
        _proj_body,
        grid=(N // BR,),
        in_specs=[
            pl.BlockSpec((BR, f), lambda i: (i, 0)),
            pl.BlockSpec((f, 2 * S2), lambda i: (0, 0)),
        ],
        out_specs=[
            pl.BlockSpec((BR, S2), lambda i: (i, 0)),
            pl.BlockSpec((BR, S2), lambda i: (i, 0)),
        ],
        out_shape=[
            jax.ShapeDtypeStruct((N, S2), jnp.float32),
            jax.ShapeDtypeStruct((N, S2), jnp.float32),
        ],
    )(x, wcat)


def _combine_proj_body(parts_ref, skip_ref, b_ref, wcat_ref, h_ref, skip2_ref):
    acc = parts_ref[0] + parts_ref[1] + skip_ref[...] + b_ref[...]
    g = _elu(acc)
    out1 = _elu(0.5 * (g[:, :C] + g[:, C:]))
    hs = jnp.dot(out1, wcat_ref[...], preferred_element_type=jnp.float32)
    h_ref[...] = hs[:, :S2]
    skip2_ref[...] = hs[:, S2:]


def _tc_combine_proj(parts, skip, brow, wcat):
    return pl.pallas_call(
        _combine_proj_body,
        grid=(N // BR,),
        in_specs=[
            pl.BlockSpec((NC, BR, S2), lambda i: (0, i, 0)),
            pl.BlockSpec((BR, S2), lambda i: (i, 0)),
            pl.BlockSpec((1, S2), lambda i: (0, 0)),
            pl.BlockSpec((C, 2 * S2), lambda i: (0, 0)),
        ],
        out_specs=[
            pl.BlockSpec((BR, S2), lambda i: (i, 0)),
            pl.BlockSpec((BR, S2), lambda i: (i, 0)),
        ],
        out_shape=[
            jax.ShapeDtypeStruct((N, S2), jnp.float32),
            jax.ShapeDtypeStruct((N, S2), jnp.float32),
        ],
    )(parts, skip, brow, wcat)


def _final_body(parts_ref, skip_ref, b_ref, wf_ref, bf_ref, out_ref):
    acc = parts_ref[0] + parts_ref[1] + skip_ref[...] + b_ref[...]
    g = _elu(acc)
    h = jax.nn.relu(0.5 * (g[:, :C] + g[:, C:]))
    o = jnp.dot(h, wf_ref[...], preferred_element_type=jnp.float32)
    out_ref[...] = jax.nn.relu(o + bf_ref[...])


def _tc_final(parts, skip, brow, wf, bfrow):
    n_out = wf.shape[1]
    return pl.pallas_call(
        _final_body,
        grid=(N // BR,),
        in_specs=[
            pl.BlockSpec((NC, BR, S2), lambda i: (0, i, 0)),
            pl.BlockSpec((BR, S2), lambda i: (i, 0)),
            pl.BlockSpec((1, S2), lambda i: (0, 0)),
            pl.BlockSpec((C, n_out), lambda i: (0, 0)),
            pl.BlockSpec((1, n_out), lambda i: (0, 0)),
        ],
        out_specs=pl.BlockSpec((BR, n_out), lambda i: (i, 0)),
        out_shape=jax.ShapeDtypeStruct((N, n_out), jnp.float32),
    )(parts, skip, brow, wf, bfrow)


def kernel(x, edge_index, edge_weight, W1, V1, b1, W2, V2, b2, Wf, bf):
    x = x.astype(jnp.float32)
    src = edge_index[0].astype(jnp.int32)
    dst = edge_index[1].astype(jnp.int32)
    w = edge_weight.astype(jnp.float32)

    e = w.shape[0]
    t = NS * (SK[0] + SK[1])
    pad = t * CH - e
    if pad:
        src = jnp.concatenate([src, jnp.zeros((pad,), jnp.int32)])
        dst = jnp.concatenate([dst, jnp.zeros((pad,), jnp.int32)])
        w = jnp.concatenate([w, jnp.zeros((pad,), jnp.float32)])
    src3 = src.reshape(t, CH)
    dst3 = dst.reshape(t, CH)
    w3 = w.reshape(t, CH)

    wcat1 = jnp.concatenate([W1[0], W1[1], V1[0], V1[1]], axis=1)
    wcat2 = jnp.concatenate([W2[0], W2[1], V2[0], V2[1]], axis=1)
    b1r = b1.reshape(1, S2)
    b2r = b2.reshape(1, S2)

    spmm = _make_spmm()

    h1, skip1 = _tc_proj(x, wcat1)
    parts1 = spmm(src3, dst3, w3, h1)
    h2, skip2 = _tc_combine_proj(parts1, skip1, b1r, wcat2)
    parts2 = spmm(src3, dst3, w3, h2)
    out = _tc_final(parts2, skip2, b2r, Wf, bf.reshape(1, Wf.shape[1]))
    return out


# speedup vs baseline: 1.0536x; 1.0536x over previous
import functools

import jax
import jax.numpy as jnp
from jax import lax
from jax.experimental import pallas as pl
from jax.experimental.pallas import tpu as pltpu
from jax.experimental.pallas import tpu_sc as plsc

N = 10000
F = 128
C = 16
S2 = 2 * C
NC = 2
NS = 16
NW = NC * NS
CH = 128
NPAD = 10240
RPW = NPAD // NS
BR = 2000
SK = (60, 99)


NBUF = 3


def _spmm_body(src_hbm, dst_hbm, wgt_hbm, h_hbm, out_hbm,
               src0, dst0, wgt0, msg0,
               src1, dst1, wgt1, msg1,
               src2, dst2, wgt2, msg2,
               zbuf_v, acc_sh,
               isem0, isem1, isem2, gsem0, gsem1, gsem2,
               ssem0, ssem1, ssem2):
    cid = lax.axis_index("c")
    sid = lax.axis_index("s")
    base = jnp.where(cid == 0, sid * SK[0], NS * SK[0] + sid * SK[1])
    ngr = jnp.where(cid == 0, SK[0] // NBUF, SK[1] // NBUF)

    srcs = (src0, src1, src2)
    dsts = (dst0, dst1, dst2)
    wgts = (wgt0, wgt1, wgt2)
    msgs = (msg0, msg1, msg2)
    isems = (isem0, isem1, isem2)
    gsems = (gsem0, gsem1, gsem2)
    ssems = (ssem0, ssem1, ssem2)

    zeros16 = jnp.zeros((16,), jnp.float32)

    def zrow(i, carry):
        zbuf_v[i, pl.ds(0, 16)] = zeros16
        zbuf_v[i, pl.ds(16, 16)] = zeros16
        return carry

    lax.fori_loop(0, RPW, zrow, 0, unroll=8)
    pltpu.sync_copy(zbuf_v, acc_sh.at[pl.ds(sid * RPW, RPW)])
    plsc.subcore_barrier()

    def fire_idx(j, b):
        pltpu.async_copy(src_hbm.at[base + j], srcs[b], isems[b])
        pltpu.async_copy(dst_hbm.at[base + j], dsts[b], isems[b])
        pltpu.async_copy(wgt_hbm.at[base + j], wgts[b], isems[b])

    def wait_idx(j, b):
        pltpu.make_async_copy(src_hbm.at[base + j], srcs[b], isems[b]).wait()
        pltpu.make_async_copy(dst_hbm.at[base + j], dsts[b], isems[b]).wait()
        pltpu.make_async_copy(wgt_hbm.at[base + j], wgts[b], isems[b]).wait()

    def wait_scatter(b):
        pltpu.make_async_copy(msgs[b], acc_sh.at[dsts[b]], ssems[b]).wait()

    def scale(b):
        wv_ = wgts[b]
        mv = msgs[b]

        def scale16(k, c2):
            wvv = wv_[pl.ds(k * 16, 16)]
            base = k * 16
            for l in range(16):
                w = wvv[l]
                mv[base + l, pl.ds(0, 16)] = mv[base + l, pl.ds(0, 16)] * w
                mv[base + l, pl.ds(16, 16)] = mv[base + l, pl.ds(16, 16)] * w
            return c2

        lax.fori_loop(0, CH // 16, scale16, 0)

    fire_idx(0, 0)
    fire_idx(1, 1)

    def group(g, carry):
        for b in range(NBUF):
            j = NBUF * g + b
            bp = (b + 2) % NBUF
            wait_idx(j, b)
            gg = pltpu.async_copy(h_hbm.at[srcs[b]], msgs[b], gsems[b])
            if b == 0:
                @pl.when(g > 0)
                def _():
                    wait_scatter(bp)
                fire_idx(j + 2, bp)
            else:
                wait_scatter(bp)

                @pl.when(g < ngr - 1)
                def _():
                    fire_idx(j + 2, bp)
            gg.wait()
            scale(b)
            pltpu.async_copy(msgs[b], acc_sh.at[dsts[b]], ssems[b], add=True)
        return carry

    lax.fori_loop(0, ngr, group, 0)
    wait_scatter(NBUF - 1)
    plsc.subcore_barrier()
    pltpu.sync_copy(acc_sh.at[pl.ds(sid * RPW, RPW)],
                    out_hbm.at[cid, pl.ds(sid * RPW, RPW)])


def _make_spmm():
    return pl.kernel(
        _spmm_body,
        out_type=jax.ShapeDtypeStruct((NC, NPAD, S2), jnp.float32),
        mesh=plsc.VectorSubcoreMesh(core_axis_name="c", subcore_axis_name="s"),
        scratch_types=(
            [pltpu.VMEM((CH,), jnp.int32),
             pltpu.VMEM((CH,), jnp.int32),
             pltpu.VMEM((CH,), jnp.float32),
             pltpu.VMEM((CH, S2), jnp.float32)] * NBUF
            + [pltpu.VMEM((RPW, S2), jnp.float32),
               pltpu.VMEM_SHARED((NPAD, S2), jnp.float32)]
            + [pltpu.SemaphoreType.DMA] * (3 * NBUF)
        ),
        compiler_params=pltpu.CompilerParams(use_tc_tiling_on_sc=False),
    )


def _elu(v):
    return jnp.where(v > 0, v, jnp.exp(jnp.minimum(v, 0.0)) - 1.0)


def _proj_body(x_ref, wcat_ref, h_ref, skip_ref):
    hs = jnp.dot(x_ref[...], wcat_ref[...], preferred_element_type=jnp.float32)
    h_ref[...] = hs[:, :S2]
    skip_ref[...] = hs[:, S2:]


def _tc_proj(x, wcat):
    f = x.shape[1]
    return pl.pallas_call(
        _proj_body,
        grid=(N // BR,),
        in_specs=[
            pl.BlockSpec((BR, f), lambda i: (i, 0)),
            pl.BlockSpec((f, 2 * S2), lambda i: (0, 0)),
        ],
        out_specs=[
            pl.BlockSpec((BR, S2), lambda i: (i, 0)),
            pl.BlockSpec((BR, S2), lambda i: (i, 0)),
        ],
        out_shape=[
            jax.ShapeDtypeStruct((N, S2), jnp.float32),
            jax.ShapeDtypeStruct((N, S2), jnp.float32),
        ],
    )(x, wcat)


def _combine_proj_body(parts_ref, skip_ref, b_ref, wcat_ref, h_ref, skip2_ref):
    acc = parts_ref[0] + parts_ref[1] + skip_ref[...] + b_ref[...]
    g = _elu(acc)
    out1 = _elu(0.5 * (g[:, :C] + g[:, C:]))
    hs = jnp.dot(out1, wcat_ref[...], preferred_element_type=jnp.float32)
    h_ref[...] = hs[:, :S2]
    skip2_ref[...] = hs[:, S2:]


def _tc_combine_proj(parts, skip, brow, wcat):
    return pl.pallas_call(
        _combine_proj_body,
        grid=(N // BR,),
        in_specs=[
            pl.BlockSpec((NC, BR, S2), lambda i: (0, i, 0)),
            pl.BlockSpec((BR, S2), lambda i: (i, 0)),
            pl.BlockSpec((1, S2), lambda i: (0, 0)),
            pl.BlockSpec((C, 2 * S2), lambda i: (0, 0)),
        ],
        out_specs=[
            pl.BlockSpec((BR, S2), lambda i: (i, 0)),
            pl.BlockSpec((BR, S2), lambda i: (i, 0)),
        ],
        out_shape=[
            jax.ShapeDtypeStruct((N, S2), jnp.float32),
            jax.ShapeDtypeStruct((N, S2), jnp.float32),
        ],
    )(parts, skip, brow, wcat)


def _final_body(parts_ref, skip_ref, b_ref, wf_ref, bf_ref, out_ref):
    acc = parts_ref[0] + parts_ref[1] + skip_ref[...] + b_ref[...]
    g = _elu(acc)
    h = jax.nn.relu(0.5 * (g[:, :C] + g[:, C:]))
    o = jnp.dot(h, wf_ref[...], preferred_element_type=jnp.float32)
    out_ref[...] = jax.nn.relu(o + bf_ref[...])


def _tc_final(parts, skip, brow, wf, bfrow):
    n_out = wf.shape[1]
    return pl.pallas_call(
        _final_body,
        grid=(N // BR,),
        in_specs=[
            pl.BlockSpec((NC, BR, S2), lambda i: (0, i, 0)),
            pl.BlockSpec((BR, S2), lambda i: (i, 0)),
            pl.BlockSpec((1, S2), lambda i: (0, 0)),
            pl.BlockSpec((C, n_out), lambda i: (0, 0)),
            pl.BlockSpec((1, n_out), lambda i: (0, 0)),
        ],
        out_specs=pl.BlockSpec((BR, n_out), lambda i: (i, 0)),
        out_shape=jax.ShapeDtypeStruct((N, n_out), jnp.float32),
    )(parts, skip, brow, wf, bfrow)


def kernel(x, edge_index, edge_weight, W1, V1, b1, W2, V2, b2, Wf, bf):
    x = x.astype(jnp.float32)
    src = edge_index[0].astype(jnp.int32)
    dst = edge_index[1].astype(jnp.int32)
    w = edge_weight.astype(jnp.float32)

    e = w.shape[0]
    t = NS * (SK[0] + SK[1])
    pad = t * CH - e
    if pad:
        src = jnp.concatenate([src, jnp.zeros((pad,), jnp.int32)])
        dst = jnp.concatenate([dst, jnp.zeros((pad,), jnp.int32)])
        w = jnp.concatenate([w, jnp.zeros((pad,), jnp.float32)])
    src3 = src.reshape(t, CH)
    dst3 = dst.reshape(t, CH)
    w3 = w.reshape(t, CH)

    wcat1 = jnp.concatenate([W1[0], W1[1], V1[0], V1[1]], axis=1)
    wcat2 = jnp.concatenate([W2[0], W2[1], V2[0], V2[1]], axis=1)
    b1r = b1.reshape(1, S2)
    b2r = b2.reshape(1, S2)

    spmm = _make_spmm()

    h1, skip1 = _tc_proj(x, wcat1)
    parts1 = spmm(src3, dst3, w3, h1)
    h2, skip2 = _tc_combine_proj(parts1, skip1, b1r, wcat2)
    parts2 = spmm(src3, dst3, w3, h2)
    out = _tc_final(parts2, skip2, b2r, Wf, bf.reshape(1, Wf.shape[1]))
    return out
